# Initial kernel scaffold; baseline (speedup 1.0000x reference)
import jax
import jax.numpy as jnp
from jax.experimental import pallas as pl
from jax.experimental.pallas import tpu as pltpu

N, D, E, R = 2048, 768, 64, 64


def _ffn_body(rules_ref, x_ref, w1_ref, b1_ref, w2_ref, b2_ref, out_ref):
    r = pl.program_id(0)

    @pl.when(r == 0)
    def _init():
        out_ref[...] = jnp.zeros_like(out_ref)

    x = x_ref[...]
    w1 = w1_ref[0]
    b1 = b1_ref[0]
    h = jax.lax.dot_general(x, w1, (((1,), (0,)), ((), ())),
                            preferred_element_type=jnp.float32)
    h = h + b1[None, :]
    h = jax.nn.gelu(h, approximate=False)
    mask = (rules_ref[0] == r)
    h = jnp.where(mask[:, None], h, 0.0)
    w2 = w2_ref[0]
    y = jax.lax.dot_general(h, w2, (((1,), (0,)), ((), ())),
                            preferred_element_type=jnp.float32)
    b2_term = jnp.where(mask[:, None], b2_ref[0][None, :], 0.0)
    out_ref[...] += y + b2_term


def kernel(x, rules, w1, b1, w2, b2):
    rules2d = rules.reshape(1, N)
    out = pl.pallas_call(
        _ffn_body,
        grid=(R,),
        in_specs=[
            pl.BlockSpec((1, N), lambda r: (0, 0)),
            pl.BlockSpec((N, D), lambda r: (0, 0)),
            pl.BlockSpec((1, D, E), lambda r: (r, 0, 0)),
            pl.BlockSpec((1, E), lambda r: (r, 0)),
            pl.BlockSpec((1, E, D), lambda r: (r, 0, 0)),
            pl.BlockSpec((1, D), lambda r: (r, 0)),
        ],
        out_specs=pl.BlockSpec((N, D), lambda r: (0, 0)),
        out_shape=jax.ShapeDtypeStruct((N, D), jnp.float32),
    )(rules2d, x, w1, b1, w2, b2)
    return out


# speedup vs baseline: 2.0693x; 2.0693x over previous
import jax
import jax.numpy as jnp
from jax.experimental import pallas as pl
from jax.experimental.pallas import tpu as pltpu

N, D, E, R = 2048, 768, 64, 64

_SQRT_HALF = 0.7071067811865476


def _gelu_exact(v):
    return 0.5 * v * (1.0 + jax.lax.erf(v * _SQRT_HALF))


def _ffn_body(rules_ref, x_ref, w1_ref, b1_ref, w2_ref, b2_ref, out_ref):
    r = pl.program_id(0)

    @pl.when(r == 0)
    def _init():
        out_ref[...] = jnp.zeros_like(out_ref)

    x = x_ref[...]
    w1 = w1_ref[0]
    b1 = b1_ref[0]
    h = jax.lax.dot_general(x, w1, (((1,), (0,)), ((), ())),
                            preferred_element_type=jnp.float32)
    h = h + b1
    h = _gelu_exact(h)
    mask = (rules_ref[...] == r)
    h = jnp.where(mask, h, 0.0)
    w2 = w2_ref[0]
    y = jax.lax.dot_general(h, w2, (((1,), (0,)), ((), ())),
                            preferred_element_type=jnp.float32)
    b2_term = jnp.where(mask, b2_ref[0], 0.0)
    out_ref[...] += y + b2_term


def kernel(x, rules, w1, b1, w2, b2):
    rules2d = rules.reshape(N, 1)
    b1r = b1.reshape(R, 1, E)
    b2r = b2.reshape(R, 1, D)
    out = pl.pallas_call(
        _ffn_body,
        grid=(R,),
        in_specs=[
            pl.BlockSpec((N, 1), lambda r: (0, 0)),
            pl.BlockSpec((N, D), lambda r: (0, 0)),
            pl.BlockSpec((1, D, E), lambda r: (r, 0, 0)),
            pl.BlockSpec((1, 1, E), lambda r: (r, 0, 0)),
            pl.BlockSpec((1, E, D), lambda r: (r, 0, 0)),
            pl.BlockSpec((1, 1, D), lambda r: (r, 0, 0)),
        ],
        out_specs=pl.BlockSpec((N, D), lambda r: (0, 0)),
        out_shape=jax.ShapeDtypeStruct((N, D), jnp.float32),
    )(rules2d, x, w1, b1r, w2, b2r)
    return out


# speedup vs baseline: 3.3340x; 1.6112x over previous
import functools

import jax
import jax.numpy as jnp
from jax import lax
from jax.experimental import pallas as pl
from jax.experimental.pallas import tpu as pltpu
from jax.experimental.pallas import tpu_sc as plsc

N, D, E, R = 2048, 768, 64, 64
B = 64
PN = N + (R - 1) * B
PN = ((PN + B - 1) // B) * B
T = PN // B

NC, NS = 2, 16
NW = NC * NS
ROWS_PER_W = N // NW

_SQRT_HALF = 0.7071067811865476


def _gelu_exact(v):
    return 0.5 * v * (1.0 + jax.lax.erf(v * _SQRT_HALF))


_RB = 128
_RG = N // _RB


def _routing_body(rules_col_ref, rules_row_ref, pos_ref, chunk_rule_ref):
    pid = pl.program_id(0)
    rules_row = rules_row_ref[...]
    r_iota0 = lax.broadcasted_iota(jnp.int32, (R, N), 0)
    eqc = (r_iota0 == rules_row).astype(jnp.int32)
    counts_col = jnp.sum(eqc, axis=1, keepdims=True)
    padded_col = ((counts_col + (B - 1)) >> 6) << 6
    tri = (lax.broadcasted_iota(jnp.int32, (R, R), 0)
           < lax.broadcasted_iota(jnp.int32, (R, R), 1)).astype(jnp.int32)
    offsets_row = jnp.sum(padded_col * tri, axis=0, keepdims=True)

    @pl.when(pid == 0)
    def _chunks():
        cb = lax.broadcasted_iota(jnp.int32, (T, R), 0) * B
        le = (offsets_row <= cb).astype(jnp.int32)
        chunk_rule_ref[...] = jnp.sum(le, axis=1, keepdims=True) - 1

    rules_blk = rules_col_ref[...]
    lane = lax.broadcasted_iota(jnp.int32, (_RB, R), 1)
    sel = (lane == rules_blk).astype(jnp.int32)
    off_tok = jnp.sum(sel * offsets_row, axis=1, keepdims=True)
    m_iota = lax.broadcasted_iota(jnp.int32, (_RB, N), 1)
    n_iota = lax.broadcasted_iota(jnp.int32, (_RB, N), 0) + pid * _RB
    same = (rules_row == rules_blk) & (m_iota < n_iota)
    rank = jnp.sum(same.astype(jnp.int32), axis=1, keepdims=True)
    pos_ref[...] = off_tok + rank


def _compute_routing(rules):
    rules_col = rules.reshape(N, 1)
    rules_row = rules.reshape(1, N)
    pos, chunk_rule = pl.pallas_call(
        _routing_body,
        grid=(_RG,),
        in_specs=[
            pl.BlockSpec((_RB, 1), lambda i: (i, 0)),
            pl.BlockSpec((1, N), lambda i: (0, 0)),
        ],
        out_specs=[
            pl.BlockSpec((_RB, 1), lambda i: (i, 0)),
            pl.BlockSpec((T, 1), lambda i: (0, 0)),
        ],
        out_shape=[
            jax.ShapeDtypeStruct((N, 1), jnp.int32),
            jax.ShapeDtypeStruct((T, 1), jnp.int32),
        ],
    )(rules_col, rules_row)
    return pos.reshape(N), chunk_rule.reshape(T)


@functools.lru_cache(maxsize=None)
def _sc_kernels():
    mesh = plsc.VectorSubcoreMesh(core_axis_name="c", subcore_axis_name="s")
    scratch = [
        pltpu.VMEM((ROWS_PER_W,), jnp.int32),
        pltpu.VMEM((ROWS_PER_W, D), jnp.float32),
        pltpu.SemaphoreType.DMA,
    ]

    @functools.partial(
        pl.kernel,
        mesh=mesh,
        out_type=jax.ShapeDtypeStruct((PN, D), jnp.float32),
        scratch_types=scratch,
    )
    def sc_scatter(x_hbm, pos_hbm, out_hbm, idx_v, rows_v, sem):
        wid = lax.axis_index("s") * NC + lax.axis_index("c")
        base = wid * ROWS_PER_W
        pltpu.sync_copy(pos_hbm.at[pl.ds(base, ROWS_PER_W)], idx_v)
        pltpu.sync_copy(x_hbm.at[pl.ds(base, ROWS_PER_W)], rows_v)
        pltpu.async_copy(rows_v, out_hbm.at[idx_v], sem).wait()

    @functools.partial(
        pl.kernel,
        mesh=mesh,
        out_type=jax.ShapeDtypeStruct((N, D), jnp.float32),
        scratch_types=scratch,
    )
    def sc_gather(y_hbm, pos_hbm, out_hbm, idx_v, rows_v, sem):
        wid = lax.axis_index("s") * NC + lax.axis_index("c")
        base = wid * ROWS_PER_W
        pltpu.sync_copy(pos_hbm.at[pl.ds(base, ROWS_PER_W)], idx_v)
        pltpu.async_copy(y_hbm.at[idx_v], rows_v, sem).wait()
        pltpu.sync_copy(rows_v, out_hbm.at[pl.ds(base, ROWS_PER_W)])

    return sc_scatter, sc_gather


def _ffn_body(chunk_rule_ref, xs_ref, w1_ref, b1_ref, w2_ref, b2_ref, y_ref):
    del chunk_rule_ref
    h = lax.dot_general(xs_ref[...], w1_ref[0], (((1,), (0,)), ((), ())),
                        preferred_element_type=jnp.float32)
    h = _gelu_exact(h + b1_ref[0])
    y = lax.dot_general(h, w2_ref[0], (((1,), (0,)), ((), ())),
                        preferred_element_type=jnp.float32)
    y_ref[...] = y + b2_ref[0]


def _grouped_ffn(xs_padded, chunk_rule, w1, b1, w2, b2):
    b1r = b1.reshape(R, 1, E)
    b2r = b2.reshape(R, 1, D)
    grid_spec = pltpu.PrefetchScalarGridSpec(
        num_scalar_prefetch=1,
        grid=(T,),
        in_specs=[
            pl.BlockSpec((B, D), lambda c, cr: (c, 0)),
            pl.BlockSpec((1, D, E), lambda c, cr: (cr[c], 0, 0)),
            pl.BlockSpec((1, 1, E), lambda c, cr: (cr[c], 0, 0)),
            pl.BlockSpec((1, E, D), lambda c, cr: (cr[c], 0, 0)),
            pl.BlockSpec((1, 1, D), lambda c, cr: (cr[c], 0, 0)),
        ],
        out_specs=pl.BlockSpec((B, D), lambda c, cr: (c, 0)),
    )
    return pl.pallas_call(
        _ffn_body,
        grid_spec=grid_spec,
        out_shape=jax.ShapeDtypeStruct((PN, D), jnp.float32),
    )(chunk_rule, xs_padded, w1, b1r, w2, b2r)


def kernel(x, rules, w1, b1, w2, b2):
    sc_scatter, sc_gather = _sc_kernels()
    pos, chunk_rule = _compute_routing(rules)
    xs_padded = sc_scatter(x, pos)
    y_padded = _grouped_ffn(xs_padded, chunk_rule, w1, b1, w2, b2)
    return sc_gather(y_padded, pos)


# speedup vs baseline: 3.7640x; 1.1290x over previous
import functools

import jax
import jax.numpy as jnp
from jax import lax
from jax.experimental import pallas as pl
from jax.experimental.pallas import tpu as pltpu
from jax.experimental.pallas import tpu_sc as plsc

N, D, E, R = 2048, 768, 64, 64
B = 64
PN = N + (R - 1) * B
PN = ((PN + B - 1) // B) * B
T = PN // B

NC, NS = 2, 16
NW = NC * NS
ROWS_PER_W = N // NW

_SQRT_HALF = 0.7071067811865476


def _gelu_exact(v):
    return 0.5 * v * (1.0 + jax.lax.erf(v * _SQRT_HALF))


_RB = 256
_RG = N // _RB


def _routing_body(rules_col_ref, rules_row_ref, pos_ref, tbl_ref):
    pid = pl.program_id(0)
    rules_row = rules_row_ref[...]
    r_iota0 = lax.broadcasted_iota(jnp.int32, (R, N), 0)
    eqc = (r_iota0 == rules_row).astype(jnp.int32)
    counts_col = jnp.sum(eqc, axis=1, keepdims=True)
    padded_col = ((counts_col + (B - 1)) >> 6) << 6
    tri = (lax.broadcasted_iota(jnp.int32, (R, R), 0)
           < lax.broadcasted_iota(jnp.int32, (R, R), 1)).astype(jnp.int32)
    offsets_row = jnp.sum(padded_col * tri, axis=0, keepdims=True)

    @pl.when(pid == 0)
    def _chunks():
        nvalid = jnp.sum(padded_col) >> 6
        c_iota = lax.broadcasted_iota(jnp.int32, (T, 1), 0)
        tgt = jnp.minimum(c_iota, nvalid)
        cb = lax.broadcasted_iota(jnp.int32, (T, R), 0) * B
        le = (offsets_row <= cb).astype(jnp.int32)
        rule = jnp.sum(le, axis=1, keepdims=True) - 1
        tbl_ref[...] = jnp.concatenate([tgt, rule], axis=0)

    rules_blk = rules_col_ref[...]
    lane = lax.broadcasted_iota(jnp.int32, (_RB, R), 1)
    sel = (lane == rules_blk).astype(jnp.int32)
    off_tok = jnp.sum(sel * offsets_row, axis=1, keepdims=True)
    m_iota = lax.broadcasted_iota(jnp.int32, (_RB, N), 1)
    n_iota = lax.broadcasted_iota(jnp.int32, (_RB, N), 0) + pid * _RB
    same = (rules_row == rules_blk) & (m_iota < n_iota)
    rank = jnp.sum(same.astype(jnp.int32), axis=1, keepdims=True)
    pos_ref[...] = off_tok + rank


def _compute_routing(rules):
    rules_col = rules.reshape(N, 1)
    rules_row = rules.reshape(1, N)
    pos, tbl = pl.pallas_call(
        _routing_body,
        grid=(_RG,),
        in_specs=[
            pl.BlockSpec((_RB, 1), lambda i: (i, 0)),
            pl.BlockSpec((1, N), lambda i: (0, 0)),
        ],
        out_specs=[
            pl.BlockSpec((_RB, 1), lambda i: (i, 0)),
            pl.BlockSpec((2 * T, 1), lambda i: (0, 0)),
        ],
        out_shape=[
            jax.ShapeDtypeStruct((N, 1), jnp.int32),
            jax.ShapeDtypeStruct((2 * T, 1), jnp.int32),
        ],
    )(rules_col, rules_row)
    return pos.reshape(N), tbl.reshape(2 * T)


@functools.lru_cache(maxsize=None)
def _sc_kernels():
    mesh = plsc.VectorSubcoreMesh(core_axis_name="c", subcore_axis_name="s")

    @functools.partial(
        pl.kernel,
        mesh=mesh,
        out_type=jax.ShapeDtypeStruct((PN, D), jnp.float32),
        scratch_types=[
            pltpu.VMEM((ROWS_PER_W,), jnp.int32),
            pltpu.VMEM((ROWS_PER_W, D), jnp.float32),
            pltpu.SemaphoreType.DMA,
        ],
    )
    def sc_scatter(x_hbm, pos_hbm, out_hbm, idx_v, rows_v, sem):
        wid = lax.axis_index("s") * NC + lax.axis_index("c")
        base = wid * ROWS_PER_W
        pltpu.sync_copy(pos_hbm.at[pl.ds(base, ROWS_PER_W)], idx_v)
        pltpu.sync_copy(x_hbm.at[pl.ds(base, ROWS_PER_W)], rows_v)
        pltpu.async_copy(rows_v, out_hbm.at[idx_v], sem).wait()

    @functools.partial(
        pl.kernel,
        mesh=mesh,
        out_type=jax.ShapeDtypeStruct((N, D), jnp.float32),
        scratch_types=[
            pltpu.VMEM((ROWS_PER_W,), jnp.int32),
            pltpu.VMEM((ROWS_PER_W, D), jnp.float32),
            pltpu.SemaphoreType.DMA,
        ],
    )
    def sc_gather(y_hbm, pos_hbm, out_hbm, idx_v, rows_v, sem):
        wid = lax.axis_index("s") * NC + lax.axis_index("c")
        base = wid * ROWS_PER_W
        pltpu.sync_copy(pos_hbm.at[pl.ds(base, ROWS_PER_W)], idx_v)
        pltpu.async_copy(y_hbm.at[idx_v], rows_v, sem).wait()
        pltpu.sync_copy(rows_v, out_hbm.at[pl.ds(base, ROWS_PER_W)])

    return sc_scatter, sc_gather


def _ffn_body(tbl_ref, xs_ref, w1_ref, b1_ref, w2_ref, b2_ref, y_ref):
    c = pl.program_id(0)

    @pl.when(tbl_ref[c] == c)
    def _compute():
        r = tbl_ref[T + c]
        xs16 = xs_ref[...].astype(jnp.bfloat16)
        h = lax.dot_general(xs16, w1_ref[r], (((1,), (0,)), ((), ())),
                            preferred_element_type=jnp.float32)
        h = _gelu_exact(h + b1_ref[r])
        y = lax.dot_general(h.astype(jnp.bfloat16), w2_ref[r],
                            (((1,), (0,)), ((), ())),
                            preferred_element_type=jnp.float32)
        y_ref[...] = y + b2_ref[r]


def _grouped_ffn(xs_padded, tbl, w1, b1, w2, b2):
    b1r = b1.reshape(R, 1, E)
    b2r = b2.reshape(R, 1, D)
    grid_spec = pltpu.PrefetchScalarGridSpec(
        num_scalar_prefetch=1,
        grid=(T,),
        in_specs=[
            pl.BlockSpec((B, D), lambda c, tbl: (tbl[c], 0)),
            pl.BlockSpec((R, D, E), lambda c, tbl: (0, 0, 0)),
            pl.BlockSpec((R, 1, E), lambda c, tbl: (0, 0, 0)),
            pl.BlockSpec((R, E, D), lambda c, tbl: (0, 0, 0)),
            pl.BlockSpec((R, 1, D), lambda c, tbl: (0, 0, 0)),
        ],
        out_specs=pl.BlockSpec((B, D), lambda c, tbl: (tbl[c], 0)),
    )
    return pl.pallas_call(
        _ffn_body,
        grid_spec=grid_spec,
        out_shape=jax.ShapeDtypeStruct((PN, D), jnp.float32),
    )(tbl, xs_padded, w1, b1r, w2, b2r)


def kernel(x, rules, w1, b1, w2, b2):
    sc_scatter, sc_gather = _sc_kernels()
    w1_16 = w1.astype(jnp.bfloat16)
    w2_16 = w2.astype(jnp.bfloat16)
    pos, tbl = _compute_routing(rules)
    xs_padded = sc_scatter(x, pos)
    y_padded = _grouped_ffn(xs_padded, tbl, w1_16, b1, w2_16, b2)
    return sc_gather(y_padded, pos)


# speedup vs baseline: 4.2729x; 1.1352x over previous
import functools

import jax
import jax.numpy as jnp
from jax import lax
from jax.experimental import pallas as pl
from jax.experimental.pallas import tpu as pltpu
from jax.experimental.pallas import tpu_sc as plsc

N, D, E, R = 2048, 768, 64, 64
B = 64
PN = N + (R - 1) * B
PN = ((PN + B - 1) // B) * B
T = PN // B
TP = T // 2
TBL = ((T + TP + 1 + 7) // 8) * 8

NC, NS = 2, 16
NW = NC * NS
ROWS_PER_W = N // NW

_SQRT_HALF = 0.7071067811865476


def _gelu_exact(v):
    return 0.5 * v * (1.0 + jax.lax.erf(v * _SQRT_HALF))


_RB = 256
_RG = N // _RB


def _routing_body(rules_col_ref, rules_row_ref, pos_ref, tbl_ref):
    pid = pl.program_id(0)
    rules_row = rules_row_ref[...]
    r_iota0 = lax.broadcasted_iota(jnp.int32, (R, N), 0)
    eqc = (r_iota0 == rules_row).astype(jnp.int32)
    counts_col = jnp.sum(eqc, axis=1, keepdims=True)
    padded_col = ((counts_col + (B - 1)) >> 6) << 6
    tri = (lax.broadcasted_iota(jnp.int32, (R, R), 0)
           < lax.broadcasted_iota(jnp.int32, (R, R), 1)).astype(jnp.int32)
    offsets_row = jnp.sum(padded_col * tri, axis=0, keepdims=True)

    @pl.when(pid == 0)
    def _chunks():
        nvalid = jnp.sum(padded_col) >> 6
        nvp = (nvalid + 1) >> 1
        p_iota = lax.broadcasted_iota(jnp.int32, (TP, 1), 0)
        pair_tgt = jnp.minimum(p_iota, nvp)
        cb = lax.broadcasted_iota(jnp.int32, (T, R), 0) * B
        le = (offsets_row <= cb).astype(jnp.int32)
        rule = jnp.sum(le, axis=1, keepdims=True) - 1
        nv = jnp.full((TBL - T - TP, 1), nvalid, jnp.int32)
        tbl_ref[...] = jnp.concatenate([pair_tgt, rule, nv], axis=0)

    rules_blk = rules_col_ref[...]
    lane = lax.broadcasted_iota(jnp.int32, (_RB, R), 1)
    sel = (lane == rules_blk).astype(jnp.int32)
    off_tok = jnp.sum(sel * offsets_row, axis=1, keepdims=True)
    m_iota = lax.broadcasted_iota(jnp.int32, (_RB, N), 1)
    n_iota = lax.broadcasted_iota(jnp.int32, (_RB, N), 0) + pid * _RB
    same = (rules_row == rules_blk) & (m_iota < n_iota)
    rank = jnp.sum(same.astype(jnp.int32), axis=1, keepdims=True)
    pos_ref[...] = off_tok + rank


def _compute_routing(rules):
    rules_col = rules.reshape(N, 1)
    rules_row = rules.reshape(1, N)
    pos, tbl = pl.pallas_call(
        _routing_body,
        grid=(_RG,),
        in_specs=[
            pl.BlockSpec((_RB, 1), lambda i: (i, 0)),
            pl.BlockSpec((1, N), lambda i: (0, 0)),
        ],
        out_specs=[
            pl.BlockSpec((_RB, 1), lambda i: (i, 0)),
            pl.BlockSpec((TBL, 1), lambda i: (0, 0)),
        ],
        out_shape=[
            jax.ShapeDtypeStruct((N, 1), jnp.int32),
            jax.ShapeDtypeStruct((TBL, 1), jnp.int32),
        ],
    )(rules_col, rules_row)
    return pos.reshape(N), tbl.reshape(TBL)


@functools.lru_cache(maxsize=None)
def _sc_kernels():
    mesh = plsc.VectorSubcoreMesh(core_axis_name="c", subcore_axis_name="s")
    scratch = [
        pltpu.VMEM((ROWS_PER_W,), jnp.int32),
        pltpu.VMEM((ROWS_PER_W, D), jnp.float32),
        pltpu.SemaphoreType.DMA,
    ]

    @functools.partial(
        pl.kernel,
        mesh=mesh,
        out_type=jax.ShapeDtypeStruct((PN, D), jnp.float32),
        scratch_types=scratch,
    )
    def sc_scatter(x_hbm, pos_hbm, out_hbm, idx_v, rows_v, sem):
        wid = lax.axis_index("s") * NC + lax.axis_index("c")
        base = wid * ROWS_PER_W
        pltpu.sync_copy(pos_hbm.at[pl.ds(base, ROWS_PER_W)], idx_v)
        pltpu.sync_copy(x_hbm.at[pl.ds(base, ROWS_PER_W)], rows_v)
        pltpu.async_copy(rows_v, out_hbm.at[idx_v], sem).wait()

    @functools.partial(
        pl.kernel,
        mesh=mesh,
        out_type=jax.ShapeDtypeStruct((N, D), jnp.float32),
        scratch_types=scratch,
    )
    def sc_gather(y_hbm, pos_hbm, out_hbm, idx_v, rows_v, sem):
        wid = lax.axis_index("s") * NC + lax.axis_index("c")
        base = wid * ROWS_PER_W
        pltpu.sync_copy(pos_hbm.at[pl.ds(base, ROWS_PER_W)], idx_v)
        pltpu.async_copy(y_hbm.at[idx_v], rows_v, sem).wait()
        pltpu.sync_copy(rows_v, out_hbm.at[pl.ds(base, ROWS_PER_W)])

    return sc_scatter, sc_gather


def _ffn_body(tbl_ref, xs_ref, w1_ref, b1_ref, w2_ref, b2_ref, y_ref):
    p = pl.program_id(0)
    for k in range(2):
        r = tbl_ref[TP + 2 * p + k]
        xs16 = xs_ref[pl.ds(k * B, B), :].astype(jnp.bfloat16)
        w1r = w1_ref[r].astype(jnp.bfloat16)
        h = lax.dot_general(xs16, w1r, (((1,), (0,)), ((), ())),
                            preferred_element_type=jnp.float32)
        h = _gelu_exact(h + b1_ref[r])
        w2r = w2_ref[r].astype(jnp.bfloat16)
        y = lax.dot_general(h.astype(jnp.bfloat16), w2r,
                            (((1,), (0,)), ((), ())),
                            preferred_element_type=jnp.float32)
        y_ref[pl.ds(k * B, B), :] = y + b2_ref[r]


def _grouped_ffn(xs_padded, tbl, w1, b1, w2, b2):
    b1r = b1.reshape(R, 1, E)
    b2r = b2.reshape(R, 1, D)
    grid_spec = pltpu.PrefetchScalarGridSpec(
        num_scalar_prefetch=1,
        grid=(TP,),
        in_specs=[
            pl.BlockSpec((2 * B, D), lambda p, tbl: (tbl[p], 0)),
            pl.BlockSpec((R, D, E), lambda p, tbl: (0, 0, 0)),
            pl.BlockSpec((R, 1, E), lambda p, tbl: (0, 0, 0)),
            pl.BlockSpec((R, E, D), lambda p, tbl: (0, 0, 0)),
            pl.BlockSpec((R, 1, D), lambda p, tbl: (0, 0, 0)),
        ],
        out_specs=pl.BlockSpec((2 * B, D), lambda p, tbl: (tbl[p], 0)),
    )
    return pl.pallas_call(
        _ffn_body,
        grid_spec=grid_spec,
        out_shape=jax.ShapeDtypeStruct((PN, D), jnp.float32),
        compiler_params=pltpu.CompilerParams(
            vmem_limit_bytes=110 * 1024 * 1024,
        ),
    )(tbl, xs_padded, w1, b1r, w2, b2r)


def kernel(x, rules, w1, b1, w2, b2):
    sc_scatter, sc_gather = _sc_kernels()
    pos, tbl = _compute_routing(rules)
    xs_padded = sc_scatter(x, pos)
    y_padded = _grouped_ffn(xs_padded, tbl, w1, b1, w2, b2)
    return sc_gather(y_padded, pos)


# speedup vs baseline: 5.0235x; 1.1756x over previous
import functools

import jax
import jax.numpy as jnp
from jax import lax
from jax.experimental import pallas as pl
from jax.experimental.pallas import tpu as pltpu
from jax.experimental.pallas import tpu_sc as plsc

N, D, E, R = 2048, 768, 64, 64
B = 64
PN = N + (R - 1) * B
PN = ((PN + B - 1) // B) * B
T = PN // B
TP = T // 2
TBL = ((T + TP + 1 + 7) // 8) * 8

NC, NS = 2, 16
NW = NC * NS
ROWS_PER_W = N // NW

_SQRT_HALF = 0.7071067811865476


def _gelu_exact(v):
    return 0.5 * v * (1.0 + jax.lax.erf(v * _SQRT_HALF))


_RB = 256
_RG = N // _RB


def _routing_body(rules_full_ref, rules_blk_ref, pos_ref, tbl_ref):
    pid = pl.program_id(0)
    rules_full = rules_full_ref[...]
    r_iota = lax.broadcasted_iota(jnp.int32, (R, N), 0)
    eqc = (r_iota == rules_full).astype(jnp.float32)
    counts_col = jnp.sum(eqc, axis=1, keepdims=True).astype(jnp.int32)
    padded_col = ((counts_col + (B - 1)) >> 6) << 6
    tri_r = (lax.broadcasted_iota(jnp.int32, (R, R), 1)
             < lax.broadcasted_iota(jnp.int32, (R, R), 0)).astype(jnp.float32)
    offsets_col = lax.dot_general(tri_r, padded_col.astype(jnp.float32),
                                  (((1,), (0,)), ((), ())),
                                  preferred_element_type=jnp.float32)

    @pl.when(pid == 0)
    def _chunks():
        nvalid = jnp.sum(padded_col) >> 6
        nvp = (nvalid + 1) >> 1
        p_iota = lax.broadcasted_iota(jnp.int32, (1, TP), 1)
        pair_tgt = jnp.minimum(p_iota, nvp)
        cb = (lax.broadcasted_iota(jnp.int32, (R, T), 1) * B).astype(jnp.float32)
        le = (offsets_col <= cb).astype(jnp.int32)
        rule = jnp.sum(le, axis=0, keepdims=True) - 1
        nv = jnp.full((1, TBL - T - TP), nvalid, jnp.int32)
        tbl_ref[...] = jnp.concatenate([pair_tgt, rule, nv], axis=1)

    rules_blk = rules_blk_ref[...]
    eqb = (lax.broadcasted_iota(jnp.int32, (R, _RB), 0)
           == rules_blk).astype(jnp.float32)
    before = (lax.broadcasted_iota(jnp.int32, (R, N), 1)
              < pid * _RB).astype(jnp.float32)
    pc_col = jnp.sum(eqc * before, axis=1, keepdims=True)
    tri_b = (lax.broadcasted_iota(jnp.int32, (_RB, _RB), 0)
             < lax.broadcasted_iota(jnp.int32, (_RB, _RB), 1)
             ).astype(jnp.float32)
    cblk = lax.dot_general(eqb, tri_b, (((1,), (0,)), ((), ())),
                           preferred_element_type=jnp.float32)
    rank = jnp.sum(eqb * (pc_col + cblk), axis=0, keepdims=True)
    off_tok = jnp.sum(eqb * offsets_col, axis=0, keepdims=True)
    pos_ref[...] = (off_tok + rank).astype(jnp.int32)


def _compute_routing(rules):
    rules_row = rules.reshape(1, N)
    pos, tbl = pl.pallas_call(
        _routing_body,
        grid=(_RG,),
        in_specs=[
            pl.BlockSpec((1, N), lambda i: (0, 0)),
            pl.BlockSpec((1, _RB), lambda i: (0, i)),
        ],
        out_specs=[
            pl.BlockSpec((1, _RB), lambda i: (0, i)),
            pl.BlockSpec((1, TBL), lambda i: (0, 0)),
        ],
        out_shape=[
            jax.ShapeDtypeStruct((1, N), jnp.int32),
            jax.ShapeDtypeStruct((1, TBL), jnp.int32),
        ],
    )(rules_row, rules_row)
    return pos.reshape(N), tbl.reshape(TBL)


@functools.lru_cache(maxsize=None)
def _sc_kernels():
    mesh = plsc.VectorSubcoreMesh(core_axis_name="c", subcore_axis_name="s")
    scratch = [
        pltpu.VMEM((ROWS_PER_W,), jnp.int32),
        pltpu.VMEM((ROWS_PER_W, D), jnp.float32),
        pltpu.SemaphoreType.DMA,
    ]

    @functools.partial(
        pl.kernel,
        mesh=mesh,
        out_type=jax.ShapeDtypeStruct((PN, D), jnp.float32),
        scratch_types=scratch,
    )
    def sc_scatter(x_hbm, pos_hbm, out_hbm, idx_v, rows_v, sem):
        wid = lax.axis_index("s") * NC + lax.axis_index("c")
        base = wid * ROWS_PER_W
        pltpu.sync_copy(pos_hbm.at[pl.ds(base, ROWS_PER_W)], idx_v)
        pltpu.sync_copy(x_hbm.at[pl.ds(base, ROWS_PER_W)], rows_v)
        pltpu.async_copy(rows_v, out_hbm.at[idx_v], sem).wait()

    @functools.partial(
        pl.kernel,
        mesh=mesh,
        out_type=jax.ShapeDtypeStruct((N, D), jnp.float32),
        scratch_types=scratch,
    )
    def sc_gather(y_hbm, pos_hbm, out_hbm, idx_v, rows_v, sem):
        wid = lax.axis_index("s") * NC + lax.axis_index("c")
        base = wid * ROWS_PER_W
        pltpu.sync_copy(pos_hbm.at[pl.ds(base, ROWS_PER_W)], idx_v)
        pltpu.async_copy(y_hbm.at[idx_v], rows_v, sem).wait()
        pltpu.sync_copy(rows_v, out_hbm.at[pl.ds(base, ROWS_PER_W)])

    return sc_scatter, sc_gather


def _ffn_body(tbl_ref, xs_ref, w1t_ref, b1_ref, w2_ref, b2_ref, y_ref):
    p = pl.program_id(0)
    for k in range(2):
        r = tbl_ref[TP + 2 * p + k]
        xs16 = xs_ref[pl.ds(k * B, B), :].astype(jnp.bfloat16)
        h = lax.dot_general(xs16, w1t_ref[r], (((1,), (1,)), ((), ())),
                            preferred_element_type=jnp.float32)
        h = _gelu_exact(h + b1_ref[pl.ds(r, 1), :])
        y = lax.dot_general(h.astype(jnp.bfloat16), w2_ref[r],
                            (((1,), (0,)), ((), ())),
                            preferred_element_type=jnp.float32)
        y_ref[pl.ds(k * B, B), :] = y + b2_ref[pl.ds(r, 1), :]


def _grouped_ffn(xs_padded, tbl, w1t16, b1, w2_16, b2):
    grid_spec = pltpu.PrefetchScalarGridSpec(
        num_scalar_prefetch=1,
        grid=(TP,),
        in_specs=[
            pl.BlockSpec((2 * B, D), lambda p, tbl: (tbl[p], 0)),
            pl.BlockSpec((R, E, D), lambda p, tbl: (0, 0, 0)),
            pl.BlockSpec((R, E), lambda p, tbl: (0, 0)),
            pl.BlockSpec((R, E, D), lambda p, tbl: (0, 0, 0)),
            pl.BlockSpec((R, D), lambda p, tbl: (0, 0)),
        ],
        out_specs=pl.BlockSpec((2 * B, D), lambda p, tbl: (tbl[p], 0)),
    )
    return pl.pallas_call(
        _ffn_body,
        grid_spec=grid_spec,
        out_shape=jax.ShapeDtypeStruct((PN, D), jnp.float32),
        compiler_params=pltpu.CompilerParams(
            vmem_limit_bytes=110 * 1024 * 1024,
        ),
    )(tbl, xs_padded, w1t16, b1, w2_16, b2)


def kernel(x, rules, w1, b1, w2, b2):
    sc_scatter, sc_gather = _sc_kernels()
    w1t16 = jnp.swapaxes(w1, 1, 2).astype(jnp.bfloat16)
    w2_16 = w2.astype(jnp.bfloat16)
    pos, tbl = _compute_routing(rules)
    xs_padded = sc_scatter(x, pos)
    y_padded = _grouped_ffn(xs_padded, tbl, w1t16, b1, w2_16, b2)
    return sc_gather(y_padded, pos)


# speedup vs baseline: 6.0163x; 1.1976x over previous
import functools

import jax
import jax.numpy as jnp
from jax import lax
from jax.experimental import pallas as pl
from jax.experimental.pallas import tpu as pltpu
from jax.experimental.pallas import tpu_sc as plsc

N, D, E, R = 2048, 768, 64, 64
B = 64
PN = N + (R - 1) * B
PN = ((PN + B - 1) // B) * B
T = PN // B
TP = T // 4
TBL = ((T + TP + 1 + 7) // 8) * 8

NC, NS = 2, 16
NW = NC * NS
ROWS_PER_W = N // NW

_SQRT_HALF = 0.7071067811865476


def _gelu_exact(v):
    return 0.5 * v * (1.0 + jax.lax.erf(v * _SQRT_HALF))


_RB = 256
_RG = N // _RB


def _routing_body(rules_full_ref, rules_blk_ref, pos_ref, tbl_ref):
    pid = pl.program_id(0)
    rules_full = rules_full_ref[...]
    r_iota = lax.broadcasted_iota(jnp.int32, (R, N), 0)
    eqc = (r_iota == rules_full).astype(jnp.float32)
    counts_col = jnp.sum(eqc, axis=1, keepdims=True).astype(jnp.int32)
    padded_col = ((counts_col + (B - 1)) >> 6) << 6
    tri_r = (lax.broadcasted_iota(jnp.int32, (R, R), 1)
             < lax.broadcasted_iota(jnp.int32, (R, R), 0)).astype(jnp.float32)
    offsets_col = lax.dot_general(tri_r, padded_col.astype(jnp.float32),
                                  (((1,), (0,)), ((), ())),
                                  preferred_element_type=jnp.float32)

    @pl.when(pid == 0)
    def _chunks():
        nvalid = jnp.sum(padded_col) >> 6
        nvp = (nvalid + 3) >> 2
        p_iota = lax.broadcasted_iota(jnp.int32, (1, TP), 1)
        pair_tgt = jnp.minimum(p_iota, nvp)
        cb = (lax.broadcasted_iota(jnp.int32, (R, T), 1) * B).astype(jnp.float32)
        le = (offsets_col <= cb).astype(jnp.int32)
        rule = jnp.sum(le, axis=0, keepdims=True) - 1
        nv = jnp.full((1, TBL - T - TP), nvalid, jnp.int32)
        tbl_ref[...] = jnp.concatenate([pair_tgt, rule, nv], axis=1)

    rules_blk = rules_blk_ref[...]
    eqb = (lax.broadcasted_iota(jnp.int32, (R, _RB), 0)
           == rules_blk).astype(jnp.float32)
    before = (lax.broadcasted_iota(jnp.int32, (R, N), 1)
              < pid * _RB).astype(jnp.float32)
    pc_col = jnp.sum(eqc * before, axis=1, keepdims=True)
    tri_b = (lax.broadcasted_iota(jnp.int32, (_RB, _RB), 0)
             < lax.broadcasted_iota(jnp.int32, (_RB, _RB), 1)
             ).astype(jnp.float32)
    cblk = lax.dot_general(eqb, tri_b, (((1,), (0,)), ((), ())),
                           preferred_element_type=jnp.float32)
    rank = jnp.sum(eqb * (pc_col + cblk), axis=0, keepdims=True)
    off_tok = jnp.sum(eqb * offsets_col, axis=0, keepdims=True)
    pos_ref[...] = (off_tok + rank).astype(jnp.int32)


def _compute_routing(rules):
    rules_row = rules.reshape(1, N)
    pos, tbl = pl.pallas_call(
        _routing_body,
        grid=(_RG,),
        in_specs=[
            pl.BlockSpec((1, N), lambda i: (0, 0)),
            pl.BlockSpec((1, _RB), lambda i: (0, i)),
        ],
        out_specs=[
            pl.BlockSpec((1, _RB), lambda i: (0, i)),
            pl.BlockSpec((1, TBL), lambda i: (0, 0)),
        ],
        out_shape=[
            jax.ShapeDtypeStruct((1, N), jnp.int32),
            jax.ShapeDtypeStruct((1, TBL), jnp.int32),
        ],
    )(rules_row, rules_row)
    return pos.reshape(N), tbl.reshape(TBL)


@functools.lru_cache(maxsize=None)
def _sc_kernels():
    mesh = plsc.VectorSubcoreMesh(core_axis_name="c", subcore_axis_name="s")
    scratch = [
        pltpu.VMEM((ROWS_PER_W,), jnp.int32),
        pltpu.VMEM((ROWS_PER_W, D), jnp.float32),
        pltpu.SemaphoreType.DMA,
    ]

    @functools.partial(
        pl.kernel,
        mesh=mesh,
        out_type=jax.ShapeDtypeStruct((PN, D), jnp.float32),
        scratch_types=scratch,
    )
    def sc_scatter(x_hbm, pos_hbm, out_hbm, idx_v, rows_v, sem):
        wid = lax.axis_index("s") * NC + lax.axis_index("c")
        base = wid * ROWS_PER_W
        pltpu.sync_copy(pos_hbm.at[pl.ds(base, ROWS_PER_W)], idx_v)
        pltpu.sync_copy(x_hbm.at[pl.ds(base, ROWS_PER_W)], rows_v)
        pltpu.async_copy(rows_v, out_hbm.at[idx_v], sem).wait()

    @functools.partial(
        pl.kernel,
        mesh=mesh,
        out_type=jax.ShapeDtypeStruct((N, D), jnp.float32),
        scratch_types=scratch,
    )
    def sc_gather(y_hbm, pos_hbm, out_hbm, idx_v, rows_v, sem):
        wid = lax.axis_index("s") * NC + lax.axis_index("c")
        base = wid * ROWS_PER_W
        pltpu.sync_copy(pos_hbm.at[pl.ds(base, ROWS_PER_W)], idx_v)
        pltpu.async_copy(y_hbm.at[idx_v], rows_v, sem).wait()
        pltpu.sync_copy(rows_v, out_hbm.at[pl.ds(base, ROWS_PER_W)])

    return sc_scatter, sc_gather


def _ffn_body(tbl_ref, xs_ref, w1t_ref, b1_ref, w2_ref, b2_ref, y_ref):
    p = pl.program_id(0)
    for k in range(4):
        r = tbl_ref[TP + 4 * p + k]
        xs16 = xs_ref[pl.ds(k * B, B), :].astype(jnp.bfloat16)
        h = lax.dot_general(xs16, w1t_ref[r], (((1,), (1,)), ((), ())),
                            preferred_element_type=jnp.float32)
        h = _gelu_exact(h + b1_ref[pl.ds(r, 1), :])
        y = lax.dot_general(h.astype(jnp.bfloat16), w2_ref[r],
                            (((1,), (0,)), ((), ())),
                            preferred_element_type=jnp.float32)
        y_ref[pl.ds(k * B, B), :] = y + b2_ref[pl.ds(r, 1), :]


def _grouped_ffn(xs_padded, tbl, w1t16, b1, w2_16, b2):
    grid_spec = pltpu.PrefetchScalarGridSpec(
        num_scalar_prefetch=1,
        grid=(TP,),
        in_specs=[
            pl.BlockSpec((4 * B, D), lambda p, tbl: (tbl[p], 0)),
            pl.BlockSpec((R, E, D), lambda p, tbl: (0, 0, 0)),
            pl.BlockSpec((R, E), lambda p, tbl: (0, 0)),
            pl.BlockSpec((R, E, D), lambda p, tbl: (0, 0, 0)),
            pl.BlockSpec((R, D), lambda p, tbl: (0, 0)),
        ],
        out_specs=pl.BlockSpec((4 * B, D), lambda p, tbl: (tbl[p], 0)),
    )
    return pl.pallas_call(
        _ffn_body,
        grid_spec=grid_spec,
        out_shape=jax.ShapeDtypeStruct((PN, D), jnp.float32),
        compiler_params=pltpu.CompilerParams(
            vmem_limit_bytes=110 * 1024 * 1024,
        ),
    )(tbl, xs_padded, w1t16, b1, w2_16, b2)


def kernel(x, rules, w1, b1, w2, b2):
    sc_scatter, sc_gather = _sc_kernels()
    w1t16 = jnp.swapaxes(w1, 1, 2).astype(jnp.bfloat16)
    w2_16 = w2.astype(jnp.bfloat16)
    pos, tbl = _compute_routing(rules)
    xs_padded = sc_scatter(x, pos)
    y_padded = _grouped_ffn(xs_padded, tbl, w1t16, b1, w2_16, b2)
    return sc_gather(y_padded, pos)


# speedup vs baseline: 6.4935x; 1.0793x over previous
import functools

import jax
import jax.numpy as jnp
from jax import lax
from jax.experimental import pallas as pl
from jax.experimental.pallas import tpu as pltpu
from jax.experimental.pallas import tpu_sc as plsc

N, D, E, R = 2048, 768, 64, 64
B = 64
PN = N + (R - 1) * B
PN = ((PN + B - 1) // B) * B
T = PN // B
TP = T // 4
TBL = ((T + TP + 1 + 7) // 8) * 8

NC, NS = 2, 16
NW = NC * NS
ROWS_PER_W = N // NW

_SQRT_HALF = 0.7071067811865476


def _gelu_exact(v):
    return 0.5 * v * (1.0 + jax.lax.erf(v * _SQRT_HALF))


_RB = 256
_RG = N // _RB


def _routing_body(rules_full_ref, rules_blk_ref, pos_ref, tbl_ref):
    pid = pl.program_id(0)
    rules_full = rules_full_ref[...]
    r_iota = lax.broadcasted_iota(jnp.int32, (R, N), 0)
    eqc = (r_iota == rules_full).astype(jnp.float32)
    counts_col = jnp.sum(eqc, axis=1, keepdims=True).astype(jnp.int32)
    padded_col = ((counts_col + (B - 1)) >> 6) << 6
    tri_r = (lax.broadcasted_iota(jnp.int32, (R, R), 1)
             < lax.broadcasted_iota(jnp.int32, (R, R), 0)).astype(jnp.float32)
    offsets_col = lax.dot_general(tri_r, padded_col.astype(jnp.float32),
                                  (((1,), (0,)), ((), ())),
                                  preferred_element_type=jnp.float32)

    @pl.when(pid == 0)
    def _chunks():
        nvalid = jnp.sum(padded_col) >> 6
        nvp = (nvalid + 3) >> 2
        p_iota = lax.broadcasted_iota(jnp.int32, (1, TP), 1)
        pair_tgt = jnp.minimum(p_iota, nvp)
        cb = (lax.broadcasted_iota(jnp.int32, (R, T), 1) * B).astype(jnp.float32)
        le = (offsets_col <= cb).astype(jnp.int32)
        rule = jnp.sum(le, axis=0, keepdims=True) - 1
        nv = jnp.full((1, TBL - T - TP), nvalid, jnp.int32)
        tbl_ref[...] = jnp.concatenate([pair_tgt, rule, nv], axis=1)

    rules_blk = rules_blk_ref[...]
    eqb = (lax.broadcasted_iota(jnp.int32, (R, _RB), 0)
           == rules_blk).astype(jnp.float32)
    before = (lax.broadcasted_iota(jnp.int32, (R, N), 1)
              < pid * _RB).astype(jnp.float32)
    pc_col = jnp.sum(eqc * before, axis=1, keepdims=True)
    tri_b = (lax.broadcasted_iota(jnp.int32, (_RB, _RB), 0)
             < lax.broadcasted_iota(jnp.int32, (_RB, _RB), 1)
             ).astype(jnp.float32)
    cblk = lax.dot_general(eqb, tri_b, (((1,), (0,)), ((), ())),
                           preferred_element_type=jnp.float32)
    rank = jnp.sum(eqb * (pc_col + cblk), axis=0, keepdims=True)
    off_tok = jnp.sum(eqb * offsets_col, axis=0, keepdims=True)
    pos_ref[...] = (off_tok + rank).astype(jnp.int32)


def _compute_routing(rules):
    rules_row = rules.reshape(1, N)
    pos, tbl = pl.pallas_call(
        _routing_body,
        grid=(_RG,),
        in_specs=[
            pl.BlockSpec((1, N), lambda i: (0, 0)),
            pl.BlockSpec((1, _RB), lambda i: (0, i)),
        ],
        out_specs=[
            pl.BlockSpec((1, _RB), lambda i: (0, i)),
            pl.BlockSpec((1, TBL), lambda i: (0, 0)),
        ],
        out_shape=[
            jax.ShapeDtypeStruct((1, N), jnp.int32),
            jax.ShapeDtypeStruct((1, TBL), jnp.int32),
        ],
    )(rules_row, rules_row)
    return pos.reshape(N), tbl.reshape(TBL)


@functools.lru_cache(maxsize=None)
def _sc_kernels():
    mesh = plsc.VectorSubcoreMesh(core_axis_name="c", subcore_axis_name="s")
    scratch = [
        pltpu.VMEM((ROWS_PER_W,), jnp.int32),
        pltpu.VMEM((ROWS_PER_W, D), jnp.float32),
        pltpu.SemaphoreType.DMA,
    ]

    @functools.partial(
        pl.kernel,
        mesh=mesh,
        out_type=jax.ShapeDtypeStruct((PN, D), jnp.float32),
        scratch_types=scratch,
    )
    def sc_scatter(x_hbm, pos_hbm, out_hbm, idx_v, rows_v, sem):
        wid = lax.axis_index("s") * NC + lax.axis_index("c")
        base = wid * ROWS_PER_W
        pltpu.sync_copy(pos_hbm.at[pl.ds(base, ROWS_PER_W)], idx_v)
        pltpu.sync_copy(x_hbm.at[pl.ds(base, ROWS_PER_W)], rows_v)
        pltpu.async_copy(rows_v, out_hbm.at[idx_v], sem).wait()

    @functools.partial(
        pl.kernel,
        mesh=mesh,
        out_type=jax.ShapeDtypeStruct((N, D), jnp.float32),
        scratch_types=scratch,
    )
    def sc_gather(y_hbm, pos_hbm, out_hbm, idx_v, rows_v, sem):
        wid = lax.axis_index("s") * NC + lax.axis_index("c")
        base = wid * ROWS_PER_W
        pltpu.sync_copy(pos_hbm.at[pl.ds(base, ROWS_PER_W)], idx_v)
        pltpu.async_copy(y_hbm.at[idx_v], rows_v, sem).wait()
        pltpu.sync_copy(rows_v, out_hbm.at[pl.ds(base, ROWS_PER_W)])

    return sc_scatter, sc_gather


def _ffn_body(tbl_ref, xs_ref, w1t_ref, b1_ref, w2_ref, b2_ref, y_ref):
    p = pl.program_id(0)
    for k in range(4):
        r = tbl_ref[TP + 4 * p + k]
        xs16 = xs_ref[pl.ds(k * B, B), :].astype(jnp.bfloat16)
        w1r = w1t_ref[r].astype(jnp.bfloat16)
        h = lax.dot_general(xs16, w1r, (((1,), (1,)), ((), ())),
                            preferred_element_type=jnp.float32)
        h = _gelu_exact(h + b1_ref[pl.ds(r, 1), :])
        y = lax.dot_general(h.astype(jnp.bfloat16), w2_ref[r],
                            (((1,), (0,)), ((), ())),
                            preferred_element_type=jnp.float32)
        y_ref[pl.ds(k * B, B), :] = y + b2_ref[pl.ds(r, 1), :]


def _grouped_ffn(xs_padded, tbl, w1t16, b1, w2_16, b2):
    grid_spec = pltpu.PrefetchScalarGridSpec(
        num_scalar_prefetch=1,
        grid=(TP,),
        in_specs=[
            pl.BlockSpec((4 * B, D), lambda p, tbl: (tbl[p], 0)),
            pl.BlockSpec((R, E, D), lambda p, tbl: (0, 0, 0)),
            pl.BlockSpec((R, E), lambda p, tbl: (0, 0)),
            pl.BlockSpec((R, E, D), lambda p, tbl: (0, 0, 0)),
            pl.BlockSpec((R, D), lambda p, tbl: (0, 0)),
        ],
        out_specs=pl.BlockSpec((4 * B, D), lambda p, tbl: (tbl[p], 0)),
    )
    return pl.pallas_call(
        _ffn_body,
        grid_spec=grid_spec,
        out_shape=jax.ShapeDtypeStruct((PN, D), jnp.float32),
        compiler_params=pltpu.CompilerParams(
            vmem_limit_bytes=110 * 1024 * 1024,
        ),
    )(tbl, xs_padded, w1t16, b1, w2_16, b2)


def kernel(x, rules, w1, b1, w2, b2):
    sc_scatter, sc_gather = _sc_kernels()
    w1t = jnp.swapaxes(w1, 1, 2)
    w2_16 = w2.astype(jnp.bfloat16)
    pos, tbl = _compute_routing(rules)
    xs_padded = sc_scatter(x, pos)
    y_padded = _grouped_ffn(xs_padded, tbl, w1t, b1, w2_16, b2)
    return sc_gather(y_padded, pos)


# speedup vs baseline: 7.3505x; 1.1320x over previous
import functools

import jax
import jax.numpy as jnp
from jax import lax
from jax.experimental import pallas as pl
from jax.experimental.pallas import tpu as pltpu
from jax.experimental.pallas import tpu_sc as plsc

N, D, E, R = 2048, 768, 64, 64
B = 64
PN = N + (R - 1) * B
PN = ((PN + B - 1) // B) * B
T = PN // B
TP = T // 8
TBL = ((T + TP + 1 + 7) // 8) * 8

NC, NS = 2, 16
NW = NC * NS
ROWS_PER_W = N // NW

_SQRT_HALF = 0.7071067811865476


def _gelu_exact(v):
    return 0.5 * v * (1.0 + jax.lax.erf(v * _SQRT_HALF))


_RB = 256
_RG = N // _RB


def _routing_body(rules_full_ref, rules_blk_ref, pos_ref, tbl_ref):
    pid = pl.program_id(0)
    rules_full = rules_full_ref[...]
    r_iota = lax.broadcasted_iota(jnp.int32, (R, N), 0)
    eqc = (r_iota == rules_full).astype(jnp.float32)
    counts_col = jnp.sum(eqc, axis=1, keepdims=True).astype(jnp.int32)
    padded_col = ((counts_col + (B - 1)) >> 6) << 6
    tri_r = (lax.broadcasted_iota(jnp.int32, (R, R), 1)
             < lax.broadcasted_iota(jnp.int32, (R, R), 0)).astype(jnp.float32)
    offsets_col = lax.dot_general(tri_r, padded_col.astype(jnp.float32),
                                  (((1,), (0,)), ((), ())),
                                  preferred_element_type=jnp.float32)

    @pl.when(pid == 0)
    def _chunks():
        nvalid = jnp.sum(padded_col) >> 6
        nvp = (nvalid + 7) >> 3
        p_iota = lax.broadcasted_iota(jnp.int32, (1, TP), 1)
        pair_tgt = jnp.minimum(p_iota, nvp)
        cb = (lax.broadcasted_iota(jnp.int32, (R, T), 1) * B).astype(jnp.float32)
        le = (offsets_col <= cb).astype(jnp.int32)
        rule = jnp.sum(le, axis=0, keepdims=True) - 1
        nv = jnp.full((1, TBL - T - TP), nvalid, jnp.int32)
        tbl_ref[...] = jnp.concatenate([pair_tgt, rule, nv], axis=1)

    rules_blk = rules_blk_ref[...]
    eqb = (lax.broadcasted_iota(jnp.int32, (R, _RB), 0)
           == rules_blk).astype(jnp.float32)
    before = (lax.broadcasted_iota(jnp.int32, (R, N), 1)
              < pid * _RB).astype(jnp.float32)
    pc_col = jnp.sum(eqc * before, axis=1, keepdims=True)
    tri_b = (lax.broadcasted_iota(jnp.int32, (_RB, _RB), 0)
             < lax.broadcasted_iota(jnp.int32, (_RB, _RB), 1)
             ).astype(jnp.float32)
    cblk = lax.dot_general(eqb, tri_b, (((1,), (0,)), ((), ())),
                           preferred_element_type=jnp.float32)
    rank = jnp.sum(eqb * (pc_col + cblk), axis=0, keepdims=True)
    off_tok = jnp.sum(eqb * offsets_col, axis=0, keepdims=True)
    pos_ref[...] = (off_tok + rank).astype(jnp.int32)


def _compute_routing(rules):
    rules_row = rules.reshape(1, N)
    pos, tbl = pl.pallas_call(
        _routing_body,
        grid=(_RG,),
        in_specs=[
            pl.BlockSpec((1, N), lambda i: (0, 0)),
            pl.BlockSpec((1, _RB), lambda i: (0, i)),
        ],
        out_specs=[
            pl.BlockSpec((1, _RB), lambda i: (0, i)),
            pl.BlockSpec((1, TBL), lambda i: (0, 0)),
        ],
        out_shape=[
            jax.ShapeDtypeStruct((1, N), jnp.int32),
            jax.ShapeDtypeStruct((1, TBL), jnp.int32),
        ],
    )(rules_row, rules_row)
    return pos.reshape(N), tbl.reshape(TBL)


@functools.lru_cache(maxsize=None)
def _sc_kernels():
    mesh = plsc.VectorSubcoreMesh(core_axis_name="c", subcore_axis_name="s")
    scratch = [
        pltpu.VMEM((ROWS_PER_W,), jnp.int32),
        pltpu.VMEM((ROWS_PER_W, D), jnp.float32),
        pltpu.SemaphoreType.DMA,
    ]

    @functools.partial(
        pl.kernel,
        mesh=mesh,
        out_type=jax.ShapeDtypeStruct((PN, D), jnp.float32),
        scratch_types=scratch,
    )
    def sc_scatter(x_hbm, pos_hbm, out_hbm, idx_v, rows_v, sem):
        wid = lax.axis_index("s") * NC + lax.axis_index("c")
        base = wid * ROWS_PER_W
        pltpu.sync_copy(pos_hbm.at[pl.ds(base, ROWS_PER_W)], idx_v)
        pltpu.sync_copy(x_hbm.at[pl.ds(base, ROWS_PER_W)], rows_v)
        pltpu.async_copy(rows_v, out_hbm.at[idx_v], sem).wait()

    @functools.partial(
        pl.kernel,
        mesh=mesh,
        out_type=jax.ShapeDtypeStruct((N, D), jnp.float32),
        scratch_types=scratch,
    )
    def sc_gather(y_hbm, pos_hbm, out_hbm, idx_v, rows_v, sem):
        wid = lax.axis_index("s") * NC + lax.axis_index("c")
        base = wid * ROWS_PER_W
        pltpu.sync_copy(pos_hbm.at[pl.ds(base, ROWS_PER_W)], idx_v)
        pltpu.async_copy(y_hbm.at[idx_v], rows_v, sem).wait()
        pltpu.sync_copy(rows_v, out_hbm.at[pl.ds(base, ROWS_PER_W)])

    return sc_scatter, sc_gather


def _ffn_body(tbl_ref, xs_ref, w1t_ref, b1_ref, w2_ref, b2_ref, y_ref):
    p = pl.program_id(0)
    for k in range(8):
        r = tbl_ref[TP + 8 * p + k]
        xs16 = xs_ref[pl.ds(k * B, B), :].astype(jnp.bfloat16)
        w1r = w1t_ref[r].astype(jnp.bfloat16)
        h = lax.dot_general(xs16, w1r, (((1,), (1,)), ((), ())),
                            preferred_element_type=jnp.float32)
        h = _gelu_exact(h + b1_ref[pl.ds(r, 1), :])
        y = lax.dot_general(h.astype(jnp.bfloat16), w2_ref[r],
                            (((1,), (0,)), ((), ())),
                            preferred_element_type=jnp.float32)
        y_ref[pl.ds(k * B, B), :] = y + b2_ref[pl.ds(r, 1), :]


def _grouped_ffn(xs_padded, tbl, w1t16, b1, w2_16, b2):
    grid_spec = pltpu.PrefetchScalarGridSpec(
        num_scalar_prefetch=1,
        grid=(TP,),
        in_specs=[
            pl.BlockSpec((8 * B, D), lambda p, tbl: (tbl[p], 0)),
            pl.BlockSpec((R, E, D), lambda p, tbl: (0, 0, 0)),
            pl.BlockSpec((R, E), lambda p, tbl: (0, 0)),
            pl.BlockSpec((R, E, D), lambda p, tbl: (0, 0, 0)),
            pl.BlockSpec((R, D), lambda p, tbl: (0, 0)),
        ],
        out_specs=pl.BlockSpec((8 * B, D), lambda p, tbl: (tbl[p], 0)),
    )
    return pl.pallas_call(
        _ffn_body,
        grid_spec=grid_spec,
        out_shape=jax.ShapeDtypeStruct((PN, D), jnp.float32),
        compiler_params=pltpu.CompilerParams(
            vmem_limit_bytes=110 * 1024 * 1024,
        ),
    )(tbl, xs_padded, w1t16, b1, w2_16, b2)


def kernel(x, rules, w1, b1, w2, b2):
    sc_scatter, sc_gather = _sc_kernels()
    w1t = jnp.swapaxes(w1, 1, 2)
    w2_16 = w2.astype(jnp.bfloat16)
    pos, tbl = _compute_routing(rules)
    xs_padded = sc_scatter(x, pos)
    y_padded = _grouped_ffn(xs_padded, tbl, w1t, b1, w2_16, b2)
    return sc_gather(y_padded, pos)


# speedup vs baseline: 7.6626x; 1.0425x over previous
import functools

import jax
import jax.numpy as jnp
from jax import lax
from jax.experimental import pallas as pl
from jax.experimental.pallas import tpu as pltpu
from jax.experimental.pallas import tpu_sc as plsc

N, D, E, R = 2048, 768, 64, 64
B = 64
PN = N + (R - 1) * B
PN = ((PN + B - 1) // B) * B
T = PN // B
TP = T // 16
TBL = ((T + TP + 1 + 7) // 8) * 8

NC, NS = 2, 16
NW = NC * NS
ROWS_PER_W = N // NW

_SQRT_HALF = 0.7071067811865476


def _gelu_exact(v):
    return 0.5 * v * (1.0 + jax.lax.erf(v * _SQRT_HALF))


_RB = 256
_RG = N // _RB


def _routing_body(rules_full_ref, rules_blk_ref, pos_ref, tbl_ref):
    pid = pl.program_id(0)
    rules_full = rules_full_ref[...]
    r_iota = lax.broadcasted_iota(jnp.int32, (R, N), 0)
    eqc = (r_iota == rules_full).astype(jnp.float32)
    counts_col = jnp.sum(eqc, axis=1, keepdims=True).astype(jnp.int32)
    padded_col = ((counts_col + (B - 1)) >> 6) << 6
    tri_r = (lax.broadcasted_iota(jnp.int32, (R, R), 1)
             < lax.broadcasted_iota(jnp.int32, (R, R), 0)).astype(jnp.float32)
    offsets_col = lax.dot_general(tri_r, padded_col.astype(jnp.float32),
                                  (((1,), (0,)), ((), ())),
                                  preferred_element_type=jnp.float32)

    @pl.when(pid == 0)
    def _chunks():
        nvalid = jnp.sum(padded_col) >> 6
        nvp = (nvalid + 15) >> 4
        p_iota = lax.broadcasted_iota(jnp.int32, (1, TP), 1)
        pair_tgt = jnp.minimum(p_iota, nvp)
        cb = (lax.broadcasted_iota(jnp.int32, (R, T), 1) * B).astype(jnp.float32)
        le = (offsets_col <= cb).astype(jnp.int32)
        rule = jnp.sum(le, axis=0, keepdims=True) - 1
        nv = jnp.full((1, TBL - T - TP), nvalid, jnp.int32)
        tbl_ref[...] = jnp.concatenate([pair_tgt, rule, nv], axis=1)

    rules_blk = rules_blk_ref[...]
    eqb = (lax.broadcasted_iota(jnp.int32, (R, _RB), 0)
           == rules_blk).astype(jnp.float32)
    before = (lax.broadcasted_iota(jnp.int32, (R, N), 1)
              < pid * _RB).astype(jnp.float32)
    pc_col = jnp.sum(eqc * before, axis=1, keepdims=True)
    tri_b = (lax.broadcasted_iota(jnp.int32, (_RB, _RB), 0)
             < lax.broadcasted_iota(jnp.int32, (_RB, _RB), 1)
             ).astype(jnp.float32)
    cblk = lax.dot_general(eqb, tri_b, (((1,), (0,)), ((), ())),
                           preferred_element_type=jnp.float32)
    rank = jnp.sum(eqb * (pc_col + cblk), axis=0, keepdims=True)
    off_tok = jnp.sum(eqb * offsets_col, axis=0, keepdims=True)
    pos_ref[...] = (off_tok + rank).astype(jnp.int32)


def _compute_routing(rules):
    rules_row = rules.reshape(1, N)
    pos, tbl = pl.pallas_call(
        _routing_body,
        grid=(_RG,),
        in_specs=[
            pl.BlockSpec((1, N), lambda i: (0, 0)),
            pl.BlockSpec((1, _RB), lambda i: (0, i)),
        ],
        out_specs=[
            pl.BlockSpec((1, _RB), lambda i: (0, i)),
            pl.BlockSpec((1, TBL), lambda i: (0, 0)),
        ],
        out_shape=[
            jax.ShapeDtypeStruct((1, N), jnp.int32),
            jax.ShapeDtypeStruct((1, TBL), jnp.int32),
        ],
    )(rules_row, rules_row)
    return pos.reshape(N), tbl.reshape(TBL)


@functools.lru_cache(maxsize=None)
def _sc_kernels():
    mesh = plsc.VectorSubcoreMesh(core_axis_name="c", subcore_axis_name="s")
    scratch = [
        pltpu.VMEM((ROWS_PER_W,), jnp.int32),
        pltpu.VMEM((ROWS_PER_W, D), jnp.float32),
        pltpu.SemaphoreType.DMA,
    ]

    @functools.partial(
        pl.kernel,
        mesh=mesh,
        out_type=jax.ShapeDtypeStruct((PN, D), jnp.float32),
        scratch_types=scratch,
    )
    def sc_scatter(x_hbm, pos_hbm, out_hbm, idx_v, rows_v, sem):
        wid = lax.axis_index("s") * NC + lax.axis_index("c")
        base = wid * ROWS_PER_W
        pltpu.sync_copy(pos_hbm.at[pl.ds(base, ROWS_PER_W)], idx_v)
        pltpu.sync_copy(x_hbm.at[pl.ds(base, ROWS_PER_W)], rows_v)
        pltpu.async_copy(rows_v, out_hbm.at[idx_v], sem).wait()

    @functools.partial(
        pl.kernel,
        mesh=mesh,
        out_type=jax.ShapeDtypeStruct((N, D), jnp.float32),
        scratch_types=scratch,
    )
    def sc_gather(y_hbm, pos_hbm, out_hbm, idx_v, rows_v, sem):
        wid = lax.axis_index("s") * NC + lax.axis_index("c")
        base = wid * ROWS_PER_W
        pltpu.sync_copy(pos_hbm.at[pl.ds(base, ROWS_PER_W)], idx_v)
        pltpu.async_copy(y_hbm.at[idx_v], rows_v, sem).wait()
        pltpu.sync_copy(rows_v, out_hbm.at[pl.ds(base, ROWS_PER_W)])

    return sc_scatter, sc_gather


def _ffn_body(tbl_ref, xs_ref, w1t_ref, b1_ref, w2_ref, b2_ref, y_ref):
    p = pl.program_id(0)
    for k in range(16):
        r = tbl_ref[TP + 16 * p + k]
        xs16 = xs_ref[pl.ds(k * B, B), :].astype(jnp.bfloat16)
        w1r = w1t_ref[r].astype(jnp.bfloat16)
        h = lax.dot_general(xs16, w1r, (((1,), (1,)), ((), ())),
                            preferred_element_type=jnp.float32)
        h = _gelu_exact(h + b1_ref[pl.ds(r, 1), :])
        y = lax.dot_general(h.astype(jnp.bfloat16), w2_ref[r],
                            (((1,), (0,)), ((), ())),
                            preferred_element_type=jnp.float32)
        y_ref[pl.ds(k * B, B), :] = y + b2_ref[pl.ds(r, 1), :]


def _grouped_ffn(xs_padded, tbl, w1t16, b1, w2_16, b2):
    grid_spec = pltpu.PrefetchScalarGridSpec(
        num_scalar_prefetch=1,
        grid=(TP,),
        in_specs=[
            pl.BlockSpec((16 * B, D), lambda p, tbl: (tbl[p], 0)),
            pl.BlockSpec((R, E, D), lambda p, tbl: (0, 0, 0)),
            pl.BlockSpec((R, E), lambda p, tbl: (0, 0)),
            pl.BlockSpec((R, E, D), lambda p, tbl: (0, 0, 0)),
            pl.BlockSpec((R, D), lambda p, tbl: (0, 0)),
        ],
        out_specs=pl.BlockSpec((16 * B, D), lambda p, tbl: (tbl[p], 0)),
    )
    return pl.pallas_call(
        _ffn_body,
        grid_spec=grid_spec,
        out_shape=jax.ShapeDtypeStruct((PN, D), jnp.float32),
        compiler_params=pltpu.CompilerParams(
            vmem_limit_bytes=110 * 1024 * 1024,
        ),
    )(tbl, xs_padded, w1t16, b1, w2_16, b2)


def kernel(x, rules, w1, b1, w2, b2):
    sc_scatter, sc_gather = _sc_kernels()
    w1t = jnp.swapaxes(w1, 1, 2)
    w2_16 = w2.astype(jnp.bfloat16)
    pos, tbl = _compute_routing(rules)
    xs_padded = sc_scatter(x, pos)
    y_padded = _grouped_ffn(xs_padded, tbl, w1t, b1, w2_16, b2)
    return sc_gather(y_padded, pos)


# speedup vs baseline: 7.9606x; 1.0389x over previous
import functools

import jax
import jax.numpy as jnp
from jax import lax
from jax.experimental import pallas as pl
from jax.experimental.pallas import tpu as pltpu
from jax.experimental.pallas import tpu_sc as plsc

N, D, E, R = 2048, 768, 64, 64
B = 64
PN = N + (R - 1) * B
PN = ((PN + B - 1) // B) * B
T = PN // B
TP = T // 16
TBL = ((T + TP + 1 + 7) // 8) * 8

NC, NS = 2, 16
NW = NC * NS
ROWS_PER_W = N // NW

_SQRT_HALF = 0.7071067811865476


def _gelu_exact(v):
    return 0.5 * v * (1.0 + jax.lax.erf(v * _SQRT_HALF))


_RB = 512
_RG = N // _RB


def _routing_body(rules_full_ref, rules_blk_ref, pos_ref, tbl_ref):
    pid = pl.program_id(0)
    rules_full = rules_full_ref[...]
    r_iota = lax.broadcasted_iota(jnp.int32, (R, N), 0)
    eqc = (r_iota == rules_full).astype(jnp.float32)
    counts_col = jnp.sum(eqc, axis=1, keepdims=True).astype(jnp.int32)
    padded_col = ((counts_col + (B - 1)) >> 6) << 6
    tri_r = (lax.broadcasted_iota(jnp.int32, (R, R), 1)
             < lax.broadcasted_iota(jnp.int32, (R, R), 0)).astype(jnp.float32)
    offsets_col = lax.dot_general(tri_r, padded_col.astype(jnp.float32),
                                  (((1,), (0,)), ((), ())),
                                  preferred_element_type=jnp.float32)

    @pl.when(pid == 0)
    def _chunks():
        nvalid = jnp.sum(padded_col) >> 6
        nvp = (nvalid + 15) >> 4
        p_iota = lax.broadcasted_iota(jnp.int32, (1, TP), 1)
        pair_tgt = jnp.minimum(p_iota, nvp)
        cb = (lax.broadcasted_iota(jnp.int32, (R, T), 1) * B).astype(jnp.float32)
        le = (offsets_col <= cb).astype(jnp.int32)
        rule = jnp.sum(le, axis=0, keepdims=True) - 1
        nv = jnp.full((1, TBL - T - TP), nvalid, jnp.int32)
        tbl_ref[...] = jnp.concatenate([pair_tgt, rule, nv], axis=1)

    rules_blk = rules_blk_ref[...]
    eqb = (lax.broadcasted_iota(jnp.int32, (R, _RB), 0)
           == rules_blk).astype(jnp.float32)
    before = (lax.broadcasted_iota(jnp.int32, (R, N), 1)
              < pid * _RB).astype(jnp.float32)
    pc_col = jnp.sum(eqc * before, axis=1, keepdims=True)
    tri_b = (lax.broadcasted_iota(jnp.int32, (_RB, _RB), 0)
             < lax.broadcasted_iota(jnp.int32, (_RB, _RB), 1)
             ).astype(jnp.float32)
    cblk = lax.dot_general(eqb, tri_b, (((1,), (0,)), ((), ())),
                           preferred_element_type=jnp.float32)
    rank = jnp.sum(eqb * (pc_col + cblk), axis=0, keepdims=True)
    off_tok = jnp.sum(eqb * offsets_col, axis=0, keepdims=True)
    pos_ref[...] = (off_tok + rank).astype(jnp.int32)


def _compute_routing(rules):
    rules_row = rules.reshape(1, N)
    pos, tbl = pl.pallas_call(
        _routing_body,
        grid=(_RG,),
        in_specs=[
            pl.BlockSpec((1, N), lambda i: (0, 0)),
            pl.BlockSpec((1, _RB), lambda i: (0, i)),
        ],
        out_specs=[
            pl.BlockSpec((1, _RB), lambda i: (0, i)),
            pl.BlockSpec((1, TBL), lambda i: (0, 0)),
        ],
        out_shape=[
            jax.ShapeDtypeStruct((1, N), jnp.int32),
            jax.ShapeDtypeStruct((1, TBL), jnp.int32),
        ],
    )(rules_row, rules_row)
    return pos.reshape(N), tbl.reshape(TBL)


@functools.lru_cache(maxsize=None)
def _sc_kernels():
    mesh = plsc.VectorSubcoreMesh(core_axis_name="c", subcore_axis_name="s")
    scratch = [
        pltpu.VMEM((ROWS_PER_W,), jnp.int32),
        pltpu.VMEM((ROWS_PER_W, D), jnp.float32),
        pltpu.SemaphoreType.DMA,
    ]

    @functools.partial(
        pl.kernel,
        mesh=mesh,
        out_type=jax.ShapeDtypeStruct((PN, D), jnp.float32),
        scratch_types=scratch,
    )
    def sc_scatter(x_hbm, pos_hbm, out_hbm, idx_v, rows_v, sem):
        wid = lax.axis_index("s") * NC + lax.axis_index("c")
        base = wid * ROWS_PER_W
        pltpu.sync_copy(pos_hbm.at[pl.ds(base, ROWS_PER_W)], idx_v)
        pltpu.sync_copy(x_hbm.at[pl.ds(base, ROWS_PER_W)], rows_v)
        pltpu.async_copy(rows_v, out_hbm.at[idx_v], sem).wait()

    @functools.partial(
        pl.kernel,
        mesh=mesh,
        out_type=jax.ShapeDtypeStruct((N, D), jnp.float32),
        scratch_types=scratch,
    )
    def sc_gather(y_hbm, pos_hbm, out_hbm, idx_v, rows_v, sem):
        wid = lax.axis_index("s") * NC + lax.axis_index("c")
        base = wid * ROWS_PER_W
        pltpu.sync_copy(pos_hbm.at[pl.ds(base, ROWS_PER_W)], idx_v)
        pltpu.async_copy(y_hbm.at[idx_v], rows_v, sem).wait()
        pltpu.sync_copy(rows_v, out_hbm.at[pl.ds(base, ROWS_PER_W)])

    return sc_scatter, sc_gather


def _ffn_body(tbl_ref, xs_ref, w1t_ref, b1_ref, w2_ref, b2_ref, y_ref):
    p = pl.program_id(0)
    for k in range(16):
        r = tbl_ref[TP + 16 * p + k]
        xs16 = xs_ref[pl.ds(k * B, B), :].astype(jnp.bfloat16)
        w1r = w1t_ref[r].astype(jnp.bfloat16)
        h = lax.dot_general(xs16, w1r, (((1,), (1,)), ((), ())),
                            preferred_element_type=jnp.float32)
        h = _gelu_exact(h + b1_ref[pl.ds(r, 1), :])
        y = lax.dot_general(h.astype(jnp.bfloat16), w2_ref[r],
                            (((1,), (0,)), ((), ())),
                            preferred_element_type=jnp.float32)
        y_ref[pl.ds(k * B, B), :] = y + b2_ref[pl.ds(r, 1), :]


def _grouped_ffn(xs_padded, tbl, w1t16, b1, w2_16, b2):
    grid_spec = pltpu.PrefetchScalarGridSpec(
        num_scalar_prefetch=1,
        grid=(TP,),
        in_specs=[
            pl.BlockSpec((16 * B, D), lambda p, tbl: (tbl[p], 0)),
            pl.BlockSpec((R, E, D), lambda p, tbl: (0, 0, 0)),
            pl.BlockSpec((R, E), lambda p, tbl: (0, 0)),
            pl.BlockSpec((R, E, D), lambda p, tbl: (0, 0, 0)),
            pl.BlockSpec((R, D), lambda p, tbl: (0, 0)),
        ],
        out_specs=pl.BlockSpec((16 * B, D), lambda p, tbl: (tbl[p], 0)),
    )
    return pl.pallas_call(
        _ffn_body,
        grid_spec=grid_spec,
        out_shape=jax.ShapeDtypeStruct((PN, D), jnp.float32),
        compiler_params=pltpu.CompilerParams(
            vmem_limit_bytes=110 * 1024 * 1024,
        ),
    )(tbl, xs_padded, w1t16, b1, w2_16, b2)


def kernel(x, rules, w1, b1, w2, b2):
    sc_scatter, sc_gather = _sc_kernels()
    w1t = jnp.swapaxes(w1, 1, 2)
    w2_16 = w2.astype(jnp.bfloat16)
    pos, tbl = _compute_routing(rules)
    xs_padded = sc_scatter(x, pos)
    y_padded = _grouped_ffn(xs_padded, tbl, w1t, b1, w2_16, b2)
    return sc_gather(y_padded, pos)


# speedup vs baseline: 8.0606x; 1.0126x over previous
import functools

import jax
import jax.numpy as jnp
from jax import lax
from jax.experimental import pallas as pl
from jax.experimental.pallas import tpu as pltpu
from jax.experimental.pallas import tpu_sc as plsc

N, D, E, R = 2048, 768, 64, 64
B = 64
PN = N + (R - 1) * B
PN = ((PN + B - 1) // B) * B
T = PN // B
TP = T // 16
TBL = ((T + TP + 1 + 7) // 8) * 8

NC, NS = 2, 16
NW = NC * NS
ROWS_PER_W = N // NW

_SQRT_HALF = 0.7071067811865476


def _gelu_exact(v):
    return 0.5 * v * (1.0 + jax.lax.erf(v * _SQRT_HALF))


_RB = 1024
_RG = N // _RB


def _routing_body(rules_full_ref, rules_blk_ref, pos_ref, tbl_ref):
    pid = pl.program_id(0)
    rules_full = rules_full_ref[...]
    r_iota = lax.broadcasted_iota(jnp.int32, (R, N), 0)
    eqc = (r_iota == rules_full).astype(jnp.float32)
    counts_col = jnp.sum(eqc, axis=1, keepdims=True).astype(jnp.int32)
    padded_col = ((counts_col + (B - 1)) >> 6) << 6
    tri_r = (lax.broadcasted_iota(jnp.int32, (R, R), 1)
             < lax.broadcasted_iota(jnp.int32, (R, R), 0)).astype(jnp.float32)
    offsets_col = lax.dot_general(tri_r, padded_col.astype(jnp.float32),
                                  (((1,), (0,)), ((), ())),
                                  preferred_element_type=jnp.float32)

    @pl.when(pid == 0)
    def _chunks():
        nvalid = jnp.sum(padded_col) >> 6
        nvp = (nvalid + 15) >> 4
        p_iota = lax.broadcasted_iota(jnp.int32, (1, TP), 1)
        pair_tgt = jnp.minimum(p_iota, nvp)
        cb = (lax.broadcasted_iota(jnp.int32, (R, T), 1) * B).astype(jnp.float32)
        le = (offsets_col <= cb).astype(jnp.int32)
        rule = jnp.sum(le, axis=0, keepdims=True) - 1
        nv = jnp.full((1, TBL - T - TP), nvalid, jnp.int32)
        tbl_ref[...] = jnp.concatenate([pair_tgt, rule, nv], axis=1)

    rules_blk = rules_blk_ref[...]
    eqb = (lax.broadcasted_iota(jnp.int32, (R, _RB), 0)
           == rules_blk).astype(jnp.float32)
    before = (lax.broadcasted_iota(jnp.int32, (R, N), 1)
              < pid * _RB).astype(jnp.float32)
    pc_col = jnp.sum(eqc * before, axis=1, keepdims=True)
    tri_b = (lax.broadcasted_iota(jnp.int32, (_RB, _RB), 0)
             < lax.broadcasted_iota(jnp.int32, (_RB, _RB), 1)
             ).astype(jnp.float32)
    cblk = lax.dot_general(eqb, tri_b, (((1,), (0,)), ((), ())),
                           preferred_element_type=jnp.float32)
    rank = jnp.sum(eqb * (pc_col + cblk), axis=0, keepdims=True)
    off_tok = jnp.sum(eqb * offsets_col, axis=0, keepdims=True)
    pos_ref[...] = (off_tok + rank).astype(jnp.int32)


def _compute_routing(rules):
    rules_row = rules.reshape(1, N)
    pos, tbl = pl.pallas_call(
        _routing_body,
        grid=(_RG,),
        in_specs=[
            pl.BlockSpec((1, N), lambda i: (0, 0)),
            pl.BlockSpec((1, _RB), lambda i: (0, i)),
        ],
        out_specs=[
            pl.BlockSpec((1, _RB), lambda i: (0, i)),
            pl.BlockSpec((1, TBL), lambda i: (0, 0)),
        ],
        out_shape=[
            jax.ShapeDtypeStruct((1, N), jnp.int32),
            jax.ShapeDtypeStruct((1, TBL), jnp.int32),
        ],
    )(rules_row, rules_row)
    return pos.reshape(N), tbl.reshape(TBL)


@functools.lru_cache(maxsize=None)
def _sc_kernels():
    mesh = plsc.VectorSubcoreMesh(core_axis_name="c", subcore_axis_name="s")
    scratch = [
        pltpu.VMEM((ROWS_PER_W,), jnp.int32),
        pltpu.VMEM((ROWS_PER_W, D), jnp.float32),
        pltpu.SemaphoreType.DMA,
    ]

    @functools.partial(
        pl.kernel,
        mesh=mesh,
        out_type=jax.ShapeDtypeStruct((PN, D), jnp.float32),
        scratch_types=scratch,
    )
    def sc_scatter(x_hbm, pos_hbm, out_hbm, idx_v, rows_v, sem):
        wid = lax.axis_index("s") * NC + lax.axis_index("c")
        base = wid * ROWS_PER_W
        pltpu.sync_copy(pos_hbm.at[pl.ds(base, ROWS_PER_W)], idx_v)
        pltpu.sync_copy(x_hbm.at[pl.ds(base, ROWS_PER_W)], rows_v)
        pltpu.async_copy(rows_v, out_hbm.at[idx_v], sem).wait()

    @functools.partial(
        pl.kernel,
        mesh=mesh,
        out_type=jax.ShapeDtypeStruct((N, D), jnp.float32),
        scratch_types=scratch,
    )
    def sc_gather(y_hbm, pos_hbm, out_hbm, idx_v, rows_v, sem):
        wid = lax.axis_index("s") * NC + lax.axis_index("c")
        base = wid * ROWS_PER_W
        pltpu.sync_copy(pos_hbm.at[pl.ds(base, ROWS_PER_W)], idx_v)
        pltpu.async_copy(y_hbm.at[idx_v], rows_v, sem).wait()
        pltpu.sync_copy(rows_v, out_hbm.at[pl.ds(base, ROWS_PER_W)])

    return sc_scatter, sc_gather


def _ffn_body(tbl_ref, xs_ref, w1t_ref, b1_ref, w2_ref, b2_ref, y_ref):
    p = pl.program_id(0)
    for k in range(16):
        r = tbl_ref[TP + 16 * p + k]
        xs16 = xs_ref[pl.ds(k * B, B), :].astype(jnp.bfloat16)
        w1r = w1t_ref[r].astype(jnp.bfloat16)
        h = lax.dot_general(xs16, w1r, (((1,), (1,)), ((), ())),
                            preferred_element_type=jnp.float32)
        h = _gelu_exact(h + b1_ref[pl.ds(r, 1), :])
        y = lax.dot_general(h.astype(jnp.bfloat16), w2_ref[r],
                            (((1,), (0,)), ((), ())),
                            preferred_element_type=jnp.float32)
        y_ref[pl.ds(k * B, B), :] = y + b2_ref[pl.ds(r, 1), :]


def _grouped_ffn(xs_padded, tbl, w1t16, b1, w2_16, b2):
    grid_spec = pltpu.PrefetchScalarGridSpec(
        num_scalar_prefetch=1,
        grid=(TP,),
        in_specs=[
            pl.BlockSpec((16 * B, D), lambda p, tbl: (tbl[p], 0)),
            pl.BlockSpec((R, E, D), lambda p, tbl: (0, 0, 0)),
            pl.BlockSpec((R, E), lambda p, tbl: (0, 0)),
            pl.BlockSpec((R, E, D), lambda p, tbl: (0, 0, 0)),
            pl.BlockSpec((R, D), lambda p, tbl: (0, 0)),
        ],
        out_specs=pl.BlockSpec((16 * B, D), lambda p, tbl: (tbl[p], 0)),
    )
    return pl.pallas_call(
        _ffn_body,
        grid_spec=grid_spec,
        out_shape=jax.ShapeDtypeStruct((PN, D), jnp.float32),
        compiler_params=pltpu.CompilerParams(
            vmem_limit_bytes=110 * 1024 * 1024,
        ),
    )(tbl, xs_padded, w1t16, b1, w2_16, b2)


def kernel(x, rules, w1, b1, w2, b2):
    sc_scatter, sc_gather = _sc_kernels()
    w1t = jnp.swapaxes(w1, 1, 2)
    w2_16 = w2.astype(jnp.bfloat16)
    pos, tbl = _compute_routing(rules)
    xs_padded = sc_scatter(x, pos)
    y_padded = _grouped_ffn(xs_padded, tbl, w1t, b1, w2_16, b2)
    return sc_gather(y_padded, pos)
